# baseline (speedup 1.0000x reference)
import jax
import jax.numpy as jnp
from jax.experimental import pallas as pl
from jax.experimental.pallas import tpu as pltpu

_CROP_TOP = 40


def _make_body(C, R, S, W, valid_rows):
    def _body(x_ref, m_ref, we_ref, wd_ref, bd_ref, o_ref):
        B = x_ref.shape[0]
        m4 = m_ref[...].reshape(S, R, W)
        we = we_ref[...]
        wd = wd_ref[...]
        bd = bd_ref[...]
        ones = jnp.ones((8, W), jnp.bfloat16)
        CK = min(256, W)
        NQ = W // CK
        D = 4
        msks = {}
        for s in range(S):
            msk = m4[s]
            if s * R < valid_rows < (s + 1) * R:
                keep = (jax.lax.broadcasted_iota(jnp.int32, (R, W), 0)
                        < valid_rows - s * R).astype(jnp.float32)
                msk = msk * keep
            msks[s] = msk
        for b in range(B):
            x4 = x_ref[b].reshape(C, S, R, W)
            for s in range(S):
                if s * R >= valid_rows:
                    o_ref[b, :, s * R:(s + 1) * R, :] = jnp.zeros(
                        (C, R, W), jnp.float32)
            chunks = [(s, q) for s in range(S) if s * R < valid_rows
                      for q in range(NQ)]
            xp = {}
            hq = {}
            for i in range(len(chunks) + D):
                if i < len(chunks):
                    s, q = chunks[i]
                    if q == 0:
                        xs = x4[:, s].reshape(C * R, W).astype(jnp.bfloat16)
                        xp[s] = jnp.concatenate([xs, ones], axis=0)
                    h = jnp.dot(we, xp[s][:, q * CK:(q + 1) * CK],
                                preferred_element_type=jnp.float32)
                    hq[i] = jnp.maximum(h.astype(jnp.bfloat16), 0)
                j = i - D
                if j >= 0:
                    sj, qj = chunks[j]
                    d = jnp.dot(wd, hq.pop(j),
                                preferred_element_type=jnp.float32)
                    lo = qj * CK
                    o_ref[b, :, sj * R:(sj + 1) * R, lo:lo + CK] = (
                        d + bd).reshape(C, R, CK) * msks[sj][None, :,
                                                             lo:lo + CK]

    return _body


def kernel(X, mask2d, g_w, g_b, ae_w1, ae_b1, ae_w2, ae_b2):
    N, C, H, W = X.shape
    Hd = ae_w1.shape[1]

    hp = jax.lax.Precision.HIGHEST
    w_enc = jnp.dot(ae_w1.T, g_w.T, precision=hp)
    b_enc = (jnp.dot(ae_w1.T, g_b.reshape(-1, 1), precision=hp)
             + ae_b1.reshape(-1, 1))

    R = 16
    eye = jnp.eye(R, dtype=jnp.float32)
    we_k = jnp.kron(w_enc, eye)
    be_col = jnp.repeat(b_enc.reshape(-1), R).reshape(-1, 1)
    we_k = jnp.concatenate(
        [we_k, be_col, jnp.zeros((Hd * R, 7), jnp.float32)], axis=1)
    we_k = we_k.astype(jnp.bfloat16)
    wd_k = jnp.kron(ae_w2.T, eye).astype(jnp.bfloat16)
    bd_k = jnp.repeat(ae_b2.reshape(-1), R).reshape(-1, 1)

    S = H // R
    valid_rows = H - _CROP_TOP
    NB = 1

    out = pl.pallas_call(
        _make_body(C, R, S, W, valid_rows),
        grid=(N // NB,),
        in_specs=[
            pl.BlockSpec((NB, C, H, W), lambda n: (n, 0, 0, 0)),
            pl.BlockSpec((H, W), lambda n: (0, 0)),
            pl.BlockSpec((Hd * R, C * R + 8), lambda n: (0, 0)),
            pl.BlockSpec((C * R, Hd * R), lambda n: (0, 0)),
            pl.BlockSpec((C * R, 1), lambda n: (0, 0)),
        ],
        out_specs=pl.BlockSpec((NB, C, H, W), lambda n: (n, 0, 0, 0)),
        out_shape=jax.ShapeDtypeStruct((N, C, H, W), jnp.float32),
        compiler_params=pltpu.CompilerParams(
            dimension_semantics=("parallel",)),
    )(X, mask2d.astype(jnp.float32), we_k, wd_k, bd_k)
    return out


# speedup vs baseline: 1.0497x; 1.0497x over previous
import jax
import jax.numpy as jnp
from jax.experimental import pallas as pl
from jax.experimental.pallas import tpu as pltpu

_CROP_TOP = 40


def _make_body(C, R, S, W, valid_rows):
    def _body(x_ref, m_ref, we_ref, wd_ref, bd_ref, o_ref):
        B = x_ref.shape[0]
        m4 = m_ref[...].reshape(S, R, W)
        we = we_ref[...]
        wd = wd_ref[...]
        bd = bd_ref[...]
        ones = jnp.ones((8, W), jnp.bfloat16)
        CK = min(256, W)
        NQ = W // CK
        D = 4
        msks = {}
        for s in range(S):
            msk = m4[s]
            if s * R < valid_rows < (s + 1) * R:
                keep = (jax.lax.broadcasted_iota(jnp.int32, (R, W), 0)
                        < valid_rows - s * R).astype(jnp.float32)
                msk = msk * keep
            msks[s] = msk
        for b in range(B):
            x4 = x_ref[b].reshape(C, S, R, W)
            for s in range(S):
                if s * R >= valid_rows:
                    o_ref[b, :, s * R:(s + 1) * R, :] = jnp.zeros(
                        (C, R, W), jnp.float32)
            chunks = [(s, q) for s in range(S) if s * R < valid_rows
                      for q in range(NQ)]
            xp = {}
            hq = {}
            for i in range(len(chunks) + D):
                if i < len(chunks):
                    s, q = chunks[i]
                    if q == 0:
                        xs = x4[:, s].reshape(C * R, W).astype(jnp.bfloat16)
                        xp[s] = jnp.concatenate([xs, ones], axis=0)
                    h = jnp.dot(we, xp[s][:, q * CK:(q + 1) * CK],
                                preferred_element_type=jnp.float32)
                    hq[i] = jnp.maximum(h.astype(jnp.bfloat16), 0)
                j = i - D
                if j >= 0:
                    sj, qj = chunks[j]
                    d = jnp.dot(wd, hq.pop(j),
                                preferred_element_type=jnp.float32)
                    lo = qj * CK
                    o_ref[b, :, sj * R:(sj + 1) * R, lo:lo + CK] = (
                        d + bd).reshape(C, R, CK) * msks[sj][None, :,
                                                             lo:lo + CK]

    return _body


def kernel(X, mask2d, g_w, g_b, ae_w1, ae_b1, ae_w2, ae_b2):
    N, C, H, W = X.shape
    Hd = ae_w1.shape[1]

    hp = jax.lax.Precision.HIGHEST
    w_enc = jnp.dot(ae_w1.T, g_w.T, precision=hp)
    b_enc = (jnp.dot(ae_w1.T, g_b.reshape(-1, 1), precision=hp)
             + ae_b1.reshape(-1, 1))

    R = 16
    eye = jnp.eye(R, dtype=jnp.float32)
    we_k = jnp.kron(w_enc, eye)
    be_col = jnp.repeat(b_enc.reshape(-1), R).reshape(-1, 1)
    we_k = jnp.concatenate(
        [we_k, be_col, jnp.zeros((Hd * R, 7), jnp.float32)], axis=1)
    we_k = we_k.astype(jnp.bfloat16)
    wd_k = jnp.kron(ae_w2.T, eye).astype(jnp.bfloat16)
    bd_k = jnp.repeat(ae_b2.reshape(-1), R).reshape(-1, 1)

    S = H // R
    valid_rows = H - _CROP_TOP
    NB = 2 if N % 2 == 0 else 1

    out = pl.pallas_call(
        _make_body(C, R, S, W, valid_rows),
        grid=(N // NB,),
        in_specs=[
            pl.BlockSpec((NB, C, H, W), lambda n: (n, 0, 0, 0)),
            pl.BlockSpec((H, W), lambda n: (0, 0)),
            pl.BlockSpec((Hd * R, C * R + 8), lambda n: (0, 0)),
            pl.BlockSpec((C * R, Hd * R), lambda n: (0, 0)),
            pl.BlockSpec((C * R, 1), lambda n: (0, 0)),
        ],
        out_specs=pl.BlockSpec((NB, C, H, W), lambda n: (n, 0, 0, 0)),
        out_shape=jax.ShapeDtypeStruct((N, C, H, W), jnp.float32),
        compiler_params=pltpu.CompilerParams(
            dimension_semantics=("parallel",)),
    )(X, mask2d.astype(jnp.float32), we_k, wd_k, bd_k)
    return out
